# Initial kernel scaffold; baseline (speedup 1.0000x reference)
import jax
import jax.numpy as jnp
from jax.experimental import pallas as pl


def kernel(x, edge_index, W_l0, b_l0, W_r0, W_l1, b_l1, W_r1):
    raise NotImplementedError("write your pallas kernel here")


# speedup vs baseline: 2.7600x; 2.7600x over previous
import jax
import jax.numpy as jnp
from jax import lax
from jax.experimental import pallas as pl
from jax.experimental.pallas import tpu as pltpu
from jax.experimental.pallas import tpu_sc as plsc

N = 10000
E = 320000
D = 128
NC, NS = 2, 16
C = 128
NCHUNK = 80
EP = NC * NS * NCHUNK * C
NP = 10240
RPT = NP // NS
ZR = 80

_MESH = dict(core_axis_name="c", subcore_axis_name="s",
             num_cores=NC, num_subcores=NS)


def _sc_agg_body(h_hbm, src_hbm, dst_hbm, zrow_hbm, acc_out,
                 src_v, dst_v, rows_v, zrow_v, acc_sh, sem):
    cid = lax.axis_index("c")
    sid = lax.axis_index("s")
    ebase = (cid * NS + sid) * NCHUNK * C

    pltpu.sync_copy(zrow_hbm, zrow_v)
    for k in range(RPT // ZR):
        pltpu.sync_copy(zrow_v, acc_sh.at[pl.ds(sid * RPT + k * ZR, ZR)])
    plsc.subcore_barrier()

    def body(j, carry):
        pltpu.sync_copy(src_hbm.at[pl.ds(ebase + j * C, C)], src_v)
        pltpu.sync_copy(dst_hbm.at[pl.ds(ebase + j * C, C)], dst_v)
        pltpu.async_copy(h_hbm.at[src_v], rows_v, sem).wait()
        pltpu.sync_copy(rows_v, acc_sh.at[dst_v], add=True)
        return carry

    lax.fori_loop(0, NCHUNK, body, 0)
    plsc.subcore_barrier()

    pltpu.sync_copy(acc_sh.at[pl.ds(sid * RPT, RPT)],
                    acc_out.at[cid, pl.ds(sid * RPT, RPT)])


_sc_agg = pl.kernel(
    _sc_agg_body,
    out_type=jax.ShapeDtypeStruct((NC, NP, D), jnp.float32),
    mesh=plsc.VectorSubcoreMesh(**_MESH),
    scratch_types=[
        pltpu.VMEM((C,), jnp.int32),
        pltpu.VMEM((C,), jnp.int32),
        pltpu.VMEM((C, D), jnp.float32),
        pltpu.VMEM((ZR, D), jnp.float32),
        pltpu.VMEM_SHARED((NP, D), jnp.float32),
        pltpu.SemaphoreType.DMA,
    ],
)


def _sc_deg_body(dst_hbm, zrow_hbm, ones_hbm, deg_out,
                 dst_v, ones_v, zrow_v, deg_sh):
    cid = lax.axis_index("c")
    sid = lax.axis_index("s")
    ebase = (cid * NS + sid) * NCHUNK * C

    pltpu.sync_copy(zrow_hbm, zrow_v)
    for k in range(RPT // ZR):
        pltpu.sync_copy(zrow_v, deg_sh.at[pl.ds(sid * RPT + k * ZR, ZR)])
    pltpu.sync_copy(ones_hbm, ones_v)
    plsc.subcore_barrier()

    def body(j, carry):
        pltpu.sync_copy(dst_hbm.at[pl.ds(ebase + j * C, C)], dst_v)
        pltpu.sync_copy(ones_v, deg_sh.at[dst_v], add=True)
        return carry

    lax.fori_loop(0, NCHUNK, body, 0)
    plsc.subcore_barrier()

    pltpu.sync_copy(deg_sh.at[pl.ds(sid * RPT, RPT)],
                    deg_out.at[cid, pl.ds(sid * RPT, RPT)])


_sc_deg = pl.kernel(
    _sc_deg_body,
    out_type=jax.ShapeDtypeStruct((NC, NP, D), jnp.float32),
    mesh=plsc.VectorSubcoreMesh(**_MESH),
    scratch_types=[
        pltpu.VMEM((C,), jnp.int32),
        pltpu.VMEM((C, D), jnp.float32),
        pltpu.VMEM((ZR, D), jnp.float32),
        pltpu.VMEM_SHARED((NP, D), jnp.float32),
    ],
)


def _tc_layer_body(h_ref, acc_ref, deg_ref, wl_ref, b_ref, wr_ref, o_ref):
    deg = deg_ref[0, :, 0:1] + deg_ref[1, :, 0:1]
    mean = (acc_ref[0] + acc_ref[1]) * (1.0 / jnp.maximum(deg, 1.0))
    o = (jnp.dot(mean, wl_ref[...], preferred_element_type=jnp.float32)
         + b_ref[...]
         + jnp.dot(h_ref[...], wr_ref[...], preferred_element_type=jnp.float32))
    o_ref[...] = jnp.maximum(o, 0.0)


_TC_R = 1000


def _tc_layer(h, acc, deg, wl_t, b, wr_t):
    return pl.pallas_call(
        _tc_layer_body,
        grid=(N // _TC_R,),
        in_specs=[
            pl.BlockSpec((_TC_R, D), lambda i: (i, 0)),
            pl.BlockSpec((NC, _TC_R, D), lambda i: (0, i, 0)),
            pl.BlockSpec((NC, _TC_R, D), lambda i: (0, i, 0)),
            pl.BlockSpec((D, D), lambda i: (0, 0)),
            pl.BlockSpec((1, D), lambda i: (0, 0)),
            pl.BlockSpec((D, D), lambda i: (0, 0)),
        ],
        out_specs=pl.BlockSpec((_TC_R, D), lambda i: (i, 0)),
        out_shape=jax.ShapeDtypeStruct((N, D), jnp.float32),
    )(h, acc, deg, wl_t, b, wr_t)


def kernel(x, edge_index, W_l0, b_l0, W_r0, W_l1, b_l1, W_r1):
    src = edge_index[0].astype(jnp.int32)
    dst = edge_index[1].astype(jnp.int32)
    src = jnp.concatenate([src, jnp.zeros((EP - E,), jnp.int32)])
    dst = jnp.concatenate([dst, jnp.full((EP - E,), N, jnp.int32)])
    zrow = jnp.zeros((ZR, D), jnp.float32)
    ones = jnp.ones((C, D), jnp.float32)

    deg = _sc_deg(dst, zrow, ones)
    acc0 = _sc_agg(x, src, dst, zrow)
    h1 = _tc_layer(x, acc0, deg, W_l0.T, b_l0.reshape(1, D), W_r0.T)
    acc1 = _sc_agg(h1, src, dst, zrow)
    out = _tc_layer(h1, acc1, deg, W_l1.T, b_l1.reshape(1, D), W_r1.T)
    return out


# speedup vs baseline: 3.3886x; 1.2278x over previous
import jax
import jax.numpy as jnp
from jax import lax
from jax.experimental import pallas as pl
from jax.experimental.pallas import tpu as pltpu
from jax.experimental.pallas import tpu_sc as plsc

N = 10000
E = 320000
D = 128
NC, NS = 2, 16
C = 128
NCHUNK = 80
EP = NC * NS * NCHUNK * C
NP = 10240
RPT = NP // NS
ZR = 80

_MESH = dict(core_axis_name="c", subcore_axis_name="s",
             num_cores=NC, num_subcores=NS)


def _sc_agg_body(h_hbm, src_hbm, dst_hbm, zrow_hbm, acc_out,
                 src_v0, dst_v0, rows_v0, src_v1, dst_v1, rows_v1,
                 zrow_v, acc_sh, sem0, sem1):
    cid = lax.axis_index("c")
    sid = lax.axis_index("s")
    ebase = (cid * NS + sid) * NCHUNK * C

    pltpu.sync_copy(zrow_hbm, zrow_v)
    for k in range(RPT // ZR):
        pltpu.sync_copy(zrow_v, acc_sh.at[pl.ds(sid * RPT + k * ZR, ZR)])
    plsc.subcore_barrier()

    bufs = ((src_v0, dst_v0, rows_v0, sem0), (src_v1, dst_v1, rows_v1, sem1))

    def fetch(j, buf):
        src_v, dst_v, rows_v, sem = buf
        pltpu.sync_copy(src_hbm.at[pl.ds(ebase + j * C, C)], src_v)
        pltpu.sync_copy(dst_hbm.at[pl.ds(ebase + j * C, C)], dst_v)
        pltpu.async_copy(h_hbm.at[src_v], rows_v, sem)

    def drain_scatter(buf):
        src_v, dst_v, rows_v, sem = buf
        pltpu.make_async_copy(h_hbm.at[src_v], rows_v, sem).wait()
        pltpu.sync_copy(rows_v, acc_sh.at[dst_v], add=True)

    fetch(0, bufs[0])

    def body(jj, carry):
        j0 = 2 * jj
        fetch(j0 + 1, bufs[1])
        drain_scatter(bufs[0])

        @pl.when(j0 + 2 < NCHUNK)
        def _():
            fetch(j0 + 2, bufs[0])

        drain_scatter(bufs[1])
        return carry

    lax.fori_loop(0, NCHUNK // 2, body, 0)
    plsc.subcore_barrier()

    pltpu.sync_copy(acc_sh.at[pl.ds(sid * RPT, RPT)],
                    acc_out.at[cid, pl.ds(sid * RPT, RPT)])


_sc_agg = pl.kernel(
    _sc_agg_body,
    out_type=jax.ShapeDtypeStruct((NC, NP, D), jnp.float32),
    mesh=plsc.VectorSubcoreMesh(**_MESH),
    scratch_types=[
        pltpu.VMEM((C,), jnp.int32),
        pltpu.VMEM((C,), jnp.int32),
        pltpu.VMEM((C, D), jnp.float32),
        pltpu.VMEM((C,), jnp.int32),
        pltpu.VMEM((C,), jnp.int32),
        pltpu.VMEM((C, D), jnp.float32),
        pltpu.VMEM((ZR, D), jnp.float32),
        pltpu.VMEM_SHARED((NP, D), jnp.float32),
        pltpu.SemaphoreType.DMA,
        pltpu.SemaphoreType.DMA,
    ],
)


def _sc_deg_body(dst_hbm, zrow_hbm, ones_hbm, deg_out,
                 dst_v, ones_v, zrow_v, deg_sh):
    cid = lax.axis_index("c")
    sid = lax.axis_index("s")
    ebase = (cid * NS + sid) * NCHUNK * C

    pltpu.sync_copy(zrow_hbm, zrow_v)
    for k in range(RPT // ZR):
        pltpu.sync_copy(zrow_v, deg_sh.at[pl.ds(sid * RPT + k * ZR, ZR)])
    pltpu.sync_copy(ones_hbm, ones_v)
    plsc.subcore_barrier()

    def body(j, carry):
        pltpu.sync_copy(dst_hbm.at[pl.ds(ebase + j * C, C)], dst_v)
        pltpu.sync_copy(ones_v, deg_sh.at[dst_v], add=True)
        return carry

    lax.fori_loop(0, NCHUNK, body, 0)
    plsc.subcore_barrier()

    pltpu.sync_copy(deg_sh.at[pl.ds(sid * RPT, RPT)],
                    deg_out.at[cid, pl.ds(sid * RPT, RPT)])


_sc_deg = pl.kernel(
    _sc_deg_body,
    out_type=jax.ShapeDtypeStruct((NC, NP, D), jnp.float32),
    mesh=plsc.VectorSubcoreMesh(**_MESH),
    scratch_types=[
        pltpu.VMEM((C,), jnp.int32),
        pltpu.VMEM((C, D), jnp.float32),
        pltpu.VMEM((ZR, D), jnp.float32),
        pltpu.VMEM_SHARED((NP, D), jnp.float32),
    ],
)


def _tc_layer_body(h_ref, acc_ref, deg_ref, wl_ref, b_ref, wr_ref, o_ref):
    deg = deg_ref[0, :, 0:1] + deg_ref[1, :, 0:1]
    mean = (acc_ref[0] + acc_ref[1]) * (1.0 / jnp.maximum(deg, 1.0))
    o = (jnp.dot(mean, wl_ref[...], preferred_element_type=jnp.float32)
         + b_ref[...]
         + jnp.dot(h_ref[...], wr_ref[...], preferred_element_type=jnp.float32))
    o_ref[...] = jnp.maximum(o, 0.0)


_TC_R = 1000


def _tc_layer(h, acc, deg, wl_t, b, wr_t):
    return pl.pallas_call(
        _tc_layer_body,
        grid=(N // _TC_R,),
        in_specs=[
            pl.BlockSpec((_TC_R, D), lambda i: (i, 0)),
            pl.BlockSpec((NC, _TC_R, D), lambda i: (0, i, 0)),
            pl.BlockSpec((NC, _TC_R, D), lambda i: (0, i, 0)),
            pl.BlockSpec((D, D), lambda i: (0, 0)),
            pl.BlockSpec((1, D), lambda i: (0, 0)),
            pl.BlockSpec((D, D), lambda i: (0, 0)),
        ],
        out_specs=pl.BlockSpec((_TC_R, D), lambda i: (i, 0)),
        out_shape=jax.ShapeDtypeStruct((N, D), jnp.float32),
    )(h, acc, deg, wl_t, b, wr_t)


def kernel(x, edge_index, W_l0, b_l0, W_r0, W_l1, b_l1, W_r1):
    src = edge_index[0].astype(jnp.int32)
    dst = edge_index[1].astype(jnp.int32)
    src = jnp.concatenate([src, jnp.zeros((EP - E,), jnp.int32)])
    dst = jnp.concatenate([dst, jnp.full((EP - E,), N, jnp.int32)])
    zrow = jnp.zeros((ZR, D), jnp.float32)
    ones = jnp.ones((C, D), jnp.float32)

    deg = _sc_deg(dst, zrow, ones)
    acc0 = _sc_agg(x, src, dst, zrow)
    h1 = _tc_layer(x, acc0, deg, W_l0.T, b_l0.reshape(1, D), W_r0.T)
    acc1 = _sc_agg(h1, src, dst, zrow)
    out = _tc_layer(h1, acc1, deg, W_l1.T, b_l1.reshape(1, D), W_r1.T)
    return out
